# Initial kernel scaffold; baseline (speedup 1.0000x reference)
import jax
import jax.numpy as jnp
from jax.experimental import pallas as pl


def kernel(x, lower_index, batch, W_down_0, W_0, W_down_1, W_1, W_down_2, W_2, lin1_w, lin1_b, lin2_w, lin2_b):
    raise NotImplementedError("write your pallas kernel here")


# speedup vs baseline: 5.5937x; 5.5937x over previous
import functools

import jax
import jax.numpy as jnp
from jax import lax
from jax.experimental import pallas as pl
from jax.experimental.pallas import tpu as pltpu
from jax.experimental.pallas import tpu_sc as plsc

_N = 10000
_E = 320000
_NC = 2
_NS = 16
_CH = 125
_OB = 16
_RPT = 624
_CHUNKS = _E // _CH
_BN = 2000
_NBLK = _N // _BN

_DOT_KW = dict(preferred_element_type=jnp.float32,
               precision=lax.Precision.HIGHEST)
_DN = (((1,), (0,)), ((), ()))


def _sc_segment_sum(hs_rows, edge_split):
    cpc = _CHUNKS // 2 if edge_split else _CHUNKS
    cpt = cpc // _NS
    mesh = plsc.VectorSubcoreMesh(core_axis_name="c", subcore_axis_name="s")

    @functools.partial(
        pl.kernel,
        out_type=jax.ShapeDtypeStruct((2 * _N, 128), jnp.float32),
        mesh=mesh,
        scratch_types=[
            pltpu.VMEM((_OB, _CH), jnp.int32),
            pltpu.VMEM((_OB, _CH), jnp.int32),
            pltpu.VMEM((_CH, 128), jnp.float32),
            pltpu.VMEM_SHARED((_N, 128), jnp.float32),
        ],
    )
    def seg(hs_hbm, src_hbm, dst_hbm, z_hbm, out_hbm, sidx, didx, rows, acc):
        c = lax.axis_index("c")
        s = lax.axis_index("s")
        r0 = s * _RPT
        pltpu.sync_copy(z_hbm.at[pl.ds(r0, _RPT)], acc.at[pl.ds(r0, _RPT)])

        @pl.when(s == _NS - 1)
        def _():
            pltpu.sync_copy(z_hbm.at[pl.ds(_NS * _RPT, _N - _NS * _RPT)],
                            acc.at[pl.ds(_NS * _RPT, _N - _NS * _RPT)])

        sbase = c * cpc + s * cpt
        dbase = (c * cpc + s * cpt) if edge_split else (s * cpt)
        plsc.subcore_barrier()

        @pl.loop(0, cpt // _OB)
        def _(t):
            pltpu.sync_copy(src_hbm.at[pl.ds(sbase + t * _OB, _OB)], sidx)
            pltpu.sync_copy(dst_hbm.at[pl.ds(dbase + t * _OB, _OB)], didx)

            @pl.loop(0, _OB)
            def _(k):
                pltpu.sync_copy(hs_hbm.at[sidx.at[k]], rows)
                pltpu.sync_copy(rows, acc.at[didx.at[k]], add=True)

        plsc.subcore_barrier()
        pltpu.sync_copy(acc.at[pl.ds(r0, _RPT)],
                        out_hbm.at[pl.ds(c * _N + r0, _RPT)])

        @pl.when(s == _NS - 1)
        def _():
            pltpu.sync_copy(
                acc.at[pl.ds(_NS * _RPT, _N - _NS * _RPT)],
                out_hbm.at[pl.ds(c * _N + _NS * _RPT, _N - _NS * _RPT)])

    def call(hs, src2, dst2, z):
        assert hs.shape == (hs_rows, 128)
        return seg(hs, src2, dst2, z)

    return call


def _tc_abs(x):
    def body(x_ref, o_ref):
        o_ref[...] = jnp.abs(x_ref[...])

    return pl.pallas_call(
        body,
        grid=(_NBLK,),
        in_specs=[pl.BlockSpec((_BN, 128), lambda i: (i, 0))],
        out_specs=pl.BlockSpec((_BN, 128), lambda i: (i, 0)),
        out_shape=jax.ShapeDtypeStruct((_N, 128), jnp.float32),
    )(x)


def _tc_layer0(aggp, h0, wd, w):
    def body(alo, ahi, h_ref, wd_ref, w_ref, o_ref):
        acc = lax.dot_general(alo[...] + ahi[...], wd_ref[...], _DN, **_DOT_KW)
        acc += lax.dot_general(h_ref[...], w_ref[...], _DN, **_DOT_KW)
        o_ref[...] = jnp.maximum(acc, 0.0)

    return pl.pallas_call(
        body,
        grid=(2, _NBLK),
        in_specs=[
            pl.BlockSpec((_BN, 128), lambda j, i: (i, 0)),
            pl.BlockSpec((_BN, 128), lambda j, i: (_NBLK + i, 0)),
            pl.BlockSpec((_BN, 128), lambda j, i: (i, 0)),
            pl.BlockSpec((128, 128), lambda j, i: (0, j)),
            pl.BlockSpec((128, 128), lambda j, i: (0, j)),
        ],
        out_specs=pl.BlockSpec((_BN, 128), lambda j, i: (j * _NBLK + i, 0)),
        out_shape=jax.ShapeDtypeStruct((2 * _N, 128), jnp.float32),
    )(aggp, aggp, h0, wd, w)


def _tc_layer(aggs, hs, wd, w):
    def body(alo, ahi, hlo, hhi, wd_ref, w_ref, o_ref):
        acc = lax.dot_general(alo[...], wd_ref[pl.ds(0, 128), :],
                              _DN, **_DOT_KW)
        acc += lax.dot_general(ahi[...], wd_ref[pl.ds(128, 128), :],
                               _DN, **_DOT_KW)
        acc += lax.dot_general(hlo[...], w_ref[pl.ds(0, 128), :],
                               _DN, **_DOT_KW)
        acc += lax.dot_general(hhi[...], w_ref[pl.ds(128, 128), :],
                               _DN, **_DOT_KW)
        o_ref[...] = jnp.maximum(acc, 0.0)

    return pl.pallas_call(
        body,
        grid=(2, _NBLK),
        in_specs=[
            pl.BlockSpec((_BN, 128), lambda j, i: (i, 0)),
            pl.BlockSpec((_BN, 128), lambda j, i: (_NBLK + i, 0)),
            pl.BlockSpec((_BN, 128), lambda j, i: (i, 0)),
            pl.BlockSpec((_BN, 128), lambda j, i: (_NBLK + i, 0)),
            pl.BlockSpec((256, 128), lambda j, i: (0, j)),
            pl.BlockSpec((256, 128), lambda j, i: (0, j)),
        ],
        out_specs=pl.BlockSpec((_BN, 128), lambda j, i: (j * _NBLK + i, 0)),
        out_shape=jax.ShapeDtypeStruct((2 * _N, 128), jnp.float32),
    )(aggs, aggs, hs, hs, wd, w)


def _tc_pool(hs3, batch2d):
    def body(h_ref, b_ref, o_ref):
        i = pl.program_id(1)

        @pl.when(i == 0)
        def _():
            o_ref[...] = jnp.zeros_like(o_ref)

        oh = (b_ref[...] == lax.broadcasted_iota(jnp.int32, (_BN, 64), 1))
        o_ref[...] += lax.dot_general(oh.astype(jnp.float32), h_ref[...],
                                      (((0,), (0,)), ((), ())), **_DOT_KW)

    return pl.pallas_call(
        body,
        grid=(2, _NBLK),
        in_specs=[
            pl.BlockSpec((_BN, 128), lambda j, i: (j * _NBLK + i, 0)),
            pl.BlockSpec((_BN, 1), lambda j, i: (i, 0)),
        ],
        out_specs=pl.BlockSpec((64, 128), lambda j, i: (0, j)),
        out_shape=jax.ShapeDtypeStruct((64, 256), jnp.float32),
    )(hs3, batch2d)


def _tc_head(pooled, w1, b1, w2, b2):
    def body(p_ref, w1_ref, b1_ref, w2_ref, b2_ref, o_ref):
        t = lax.dot_general(p_ref[...], w1_ref[...], _DN,
                            **_DOT_KW) + b1_ref[...]
        t = jnp.maximum(t, 0.0)
        o_ref[...] = lax.dot_general(t, w2_ref[...], _DN,
                                     **_DOT_KW) + b2_ref[...]

    return pl.pallas_call(
        body,
        out_shape=jax.ShapeDtypeStruct((64, 10), jnp.float32),
    )(pooled, w1, b1, w2, b2)


def kernel(x, lower_index, batch, W_down_0, W_0, W_down_1, W_1,
           W_down_2, W_2, lin1_w, lin1_b, lin2_w, lin2_b):
    src = lower_index[0]
    dst = lower_index[1]
    src2 = jnp.concatenate([src, src + _N]).reshape(2 * _CHUNKS, _CH)
    dst2 = dst.reshape(_CHUNKS, _CH)
    z128 = jnp.zeros((_N, 128), jnp.float32)
    batch2d = batch.reshape(_N, 1)
    b1 = lin1_b.reshape(1, 256)
    b2 = lin2_b.reshape(1, 10)

    seg_e = _sc_segment_sum(_N, edge_split=True)
    seg_f = _sc_segment_sum(2 * _N, edge_split=False)

    h0 = _tc_abs(x)
    a0 = seg_e(h0, src2, dst2, z128)
    hs1 = _tc_layer0(a0, h0, W_down_0, W_0)
    a1 = seg_f(hs1, src2, dst2, z128)
    hs2 = _tc_layer(a1, hs1, W_down_1, W_1)
    a2 = seg_f(hs2, src2, dst2, z128)
    hs3 = _tc_layer(a2, hs2, W_down_2, W_2)
    pooled = _tc_pool(hs3, batch2d)
    return _tc_head(pooled, lin1_w, b1, lin2_w, b2)


# speedup vs baseline: 7.9942x; 1.4292x over previous
import functools

import jax
import jax.numpy as jnp
from jax import lax
from jax.experimental import pallas as pl
from jax.experimental.pallas import tpu as pltpu
from jax.experimental.pallas import tpu_sc as plsc

_N = 10000
_E = 320000
_NC = 2
_NS = 16
_CH = 125
_OB = 16
_RPT = 624
_CHUNKS = _E // _CH
_BN = 2000
_NBLK = _N // _BN

_DOT_KW = dict(preferred_element_type=jnp.float32,
               precision=lax.Precision.HIGHEST)
_DN = (((1,), (0,)), ((), ()))


def _sc_segment_sum(hs_rows, edge_split):
    cpc = _CHUNKS // 2 if edge_split else _CHUNKS
    cpt = cpc // _NS
    mesh = plsc.VectorSubcoreMesh(core_axis_name="c", subcore_axis_name="s")

    @functools.partial(
        pl.kernel,
        out_type=jax.ShapeDtypeStruct((2 * _N, 128), jnp.float32),
        mesh=mesh,
        scratch_types=[
            pltpu.VMEM((_OB, _CH), jnp.int32),
            pltpu.VMEM((_OB, _CH), jnp.int32),
            pltpu.VMEM((_CH, 128), jnp.float32),
            pltpu.VMEM((_CH, 128), jnp.float32),
            pltpu.VMEM_SHARED((_N, 128), jnp.float32),
            pltpu.SemaphoreType.DMA,
            pltpu.SemaphoreType.DMA,
        ],
    )
    def seg(hs_hbm, src_hbm, dst_hbm, z_hbm, out_hbm,
            sidx, didx, rows0, rows1, acc, gsem0, gsem1):
        c = lax.axis_index("c")
        s = lax.axis_index("s")
        r0 = s * _RPT
        pltpu.sync_copy(z_hbm.at[pl.ds(r0, _RPT)], acc.at[pl.ds(r0, _RPT)])

        @pl.when(s == _NS - 1)
        def _():
            pltpu.sync_copy(z_hbm.at[pl.ds(_NS * _RPT, _N - _NS * _RPT)],
                            acc.at[pl.ds(_NS * _RPT, _N - _NS * _RPT)])

        sbase = c * cpc + s * cpt
        dbase = (c * cpc + s * cpt) if edge_split else (s * cpt)
        plsc.subcore_barrier()

        rows = (rows0, rows1)
        sems = (gsem0, gsem1)

        @pl.loop(0, cpt // _OB)
        def _(t):
            pltpu.sync_copy(src_hbm.at[pl.ds(sbase + t * _OB, _OB)], sidx)
            pltpu.sync_copy(dst_hbm.at[pl.ds(dbase + t * _OB, _OB)], didx)
            pend = [pltpu.async_copy(hs_hbm.at[sidx.at[0]], rows[0], sems[0]),
                    None]
            for j in range(_OB):
                if j + 1 < _OB:
                    b = (j + 1) % 2
                    pend[b] = pltpu.async_copy(hs_hbm.at[sidx.at[j + 1]],
                                               rows[b], sems[b])
                pend[j % 2].wait()
                pltpu.sync_copy(rows[j % 2], acc.at[didx.at[j]], add=True)

        plsc.subcore_barrier()
        pltpu.sync_copy(acc.at[pl.ds(r0, _RPT)],
                        out_hbm.at[pl.ds(c * _N + r0, _RPT)])

        @pl.when(s == _NS - 1)
        def _():
            pltpu.sync_copy(
                acc.at[pl.ds(_NS * _RPT, _N - _NS * _RPT)],
                out_hbm.at[pl.ds(c * _N + _NS * _RPT, _N - _NS * _RPT)])

    def call(hs, src2, dst2, z):
        assert hs.shape == (hs_rows, 128)
        return seg(hs, src2, dst2, z)

    return call


def _tc_abs(x):
    def body(x_ref, o_ref):
        o_ref[...] = jnp.abs(x_ref[...])

    return pl.pallas_call(
        body,
        grid=(_NBLK,),
        in_specs=[pl.BlockSpec((_BN, 128), lambda i: (i, 0))],
        out_specs=pl.BlockSpec((_BN, 128), lambda i: (i, 0)),
        out_shape=jax.ShapeDtypeStruct((_N, 128), jnp.float32),
    )(x)


def _tc_layer0(aggp, h0, wd, w):
    def body(alo, ahi, h_ref, wd_ref, w_ref, o_ref):
        acc = lax.dot_general(alo[...] + ahi[...], wd_ref[...], _DN, **_DOT_KW)
        acc += lax.dot_general(h_ref[...], w_ref[...], _DN, **_DOT_KW)
        o_ref[...] = jnp.maximum(acc, 0.0)

    return pl.pallas_call(
        body,
        grid=(2, _NBLK),
        in_specs=[
            pl.BlockSpec((_BN, 128), lambda j, i: (i, 0)),
            pl.BlockSpec((_BN, 128), lambda j, i: (_NBLK + i, 0)),
            pl.BlockSpec((_BN, 128), lambda j, i: (i, 0)),
            pl.BlockSpec((128, 128), lambda j, i: (0, j)),
            pl.BlockSpec((128, 128), lambda j, i: (0, j)),
        ],
        out_specs=pl.BlockSpec((_BN, 128), lambda j, i: (j * _NBLK + i, 0)),
        out_shape=jax.ShapeDtypeStruct((2 * _N, 128), jnp.float32),
    )(aggp, aggp, h0, wd, w)


def _tc_layer(aggs, hs, wd, w):
    def body(alo, ahi, hlo, hhi, wd_ref, w_ref, o_ref):
        acc = lax.dot_general(alo[...], wd_ref[pl.ds(0, 128), :],
                              _DN, **_DOT_KW)
        acc += lax.dot_general(ahi[...], wd_ref[pl.ds(128, 128), :],
                               _DN, **_DOT_KW)
        acc += lax.dot_general(hlo[...], w_ref[pl.ds(0, 128), :],
                               _DN, **_DOT_KW)
        acc += lax.dot_general(hhi[...], w_ref[pl.ds(128, 128), :],
                               _DN, **_DOT_KW)
        o_ref[...] = jnp.maximum(acc, 0.0)

    return pl.pallas_call(
        body,
        grid=(2, _NBLK),
        in_specs=[
            pl.BlockSpec((_BN, 128), lambda j, i: (i, 0)),
            pl.BlockSpec((_BN, 128), lambda j, i: (_NBLK + i, 0)),
            pl.BlockSpec((_BN, 128), lambda j, i: (i, 0)),
            pl.BlockSpec((_BN, 128), lambda j, i: (_NBLK + i, 0)),
            pl.BlockSpec((256, 128), lambda j, i: (0, j)),
            pl.BlockSpec((256, 128), lambda j, i: (0, j)),
        ],
        out_specs=pl.BlockSpec((_BN, 128), lambda j, i: (j * _NBLK + i, 0)),
        out_shape=jax.ShapeDtypeStruct((2 * _N, 128), jnp.float32),
    )(aggs, aggs, hs, hs, wd, w)


def _tc_pool(hs3, batch2d):
    def body(h_ref, b_ref, o_ref):
        i = pl.program_id(1)

        @pl.when(i == 0)
        def _():
            o_ref[...] = jnp.zeros_like(o_ref)

        oh = (b_ref[...] == lax.broadcasted_iota(jnp.int32, (_BN, 64), 1))
        o_ref[...] += lax.dot_general(oh.astype(jnp.float32), h_ref[...],
                                      (((0,), (0,)), ((), ())), **_DOT_KW)

    return pl.pallas_call(
        body,
        grid=(2, _NBLK),
        in_specs=[
            pl.BlockSpec((_BN, 128), lambda j, i: (j * _NBLK + i, 0)),
            pl.BlockSpec((_BN, 1), lambda j, i: (i, 0)),
        ],
        out_specs=pl.BlockSpec((64, 128), lambda j, i: (0, j)),
        out_shape=jax.ShapeDtypeStruct((64, 256), jnp.float32),
    )(hs3, batch2d)


def _tc_head(pooled, w1, b1, w2, b2):
    def body(p_ref, w1_ref, b1_ref, w2_ref, b2_ref, o_ref):
        t = lax.dot_general(p_ref[...], w1_ref[...], _DN,
                            **_DOT_KW) + b1_ref[...]
        t = jnp.maximum(t, 0.0)
        o_ref[...] = lax.dot_general(t, w2_ref[...], _DN,
                                     **_DOT_KW) + b2_ref[...]

    return pl.pallas_call(
        body,
        out_shape=jax.ShapeDtypeStruct((64, 10), jnp.float32),
    )(pooled, w1, b1, w2, b2)


def kernel(x, lower_index, batch, W_down_0, W_0, W_down_1, W_1,
           W_down_2, W_2, lin1_w, lin1_b, lin2_w, lin2_b):
    src = lower_index[0]
    dst = lower_index[1]
    src2 = jnp.concatenate([src, src + _N]).reshape(2 * _CHUNKS, _CH)
    dst2 = dst.reshape(_CHUNKS, _CH)
    z128 = jnp.zeros((_N, 128), jnp.float32)
    batch2d = batch.reshape(_N, 1)
    b1 = lin1_b.reshape(1, 256)
    b2 = lin2_b.reshape(1, 10)

    seg_e = _sc_segment_sum(_N, edge_split=True)
    seg_f = _sc_segment_sum(2 * _N, edge_split=False)

    h0 = _tc_abs(x)
    a0 = seg_e(h0, src2, dst2, z128)
    hs1 = _tc_layer0(a0, h0, W_down_0, W_0)
    a1 = seg_f(hs1, src2, dst2, z128)
    hs2 = _tc_layer(a1, hs1, W_down_1, W_1)
    a2 = seg_f(hs2, src2, dst2, z128)
    hs3 = _tc_layer(a2, hs2, W_down_2, W_2)
    pooled = _tc_pool(hs3, batch2d)
    return _tc_head(pooled, lin1_w, b1, lin2_w, b2)


# speedup vs baseline: 8.3986x; 1.0506x over previous
import functools

import jax
import jax.numpy as jnp
from jax import lax
from jax.experimental import pallas as pl
from jax.experimental.pallas import tpu as pltpu
from jax.experimental.pallas import tpu_sc as plsc

_N = 10000
_E = 320000
_NC = 2
_NS = 16
_CH = 125
_OB = 16
_RPT = 624
_CHUNKS = _E // _CH
_BN = 2000
_NBLK = _N // _BN

_DOT_KW = dict(preferred_element_type=jnp.float32,
               precision=lax.Precision.HIGHEST)
_DN = (((1,), (0,)), ((), ()))


def _sc_segment_sum(hs_rows, edge_split):
    cpc = _CHUNKS // 2 if edge_split else _CHUNKS
    cpt = cpc // _NS
    ob = 16 if edge_split else 32
    mesh = plsc.VectorSubcoreMesh(core_axis_name="c", subcore_axis_name="s")

    @functools.partial(
        pl.kernel,
        out_type=jax.ShapeDtypeStruct((2 * _N, 128), jnp.float32),
        mesh=mesh,
        scratch_types=[
            pltpu.VMEM((ob, _CH), jnp.int32),
            pltpu.VMEM((ob, _CH), jnp.int32),
            pltpu.VMEM((_CH, 128), jnp.float32),
            pltpu.VMEM((_CH, 128), jnp.float32),
            pltpu.VMEM_SHARED((_N, 128), jnp.float32),
            pltpu.SemaphoreType.DMA,
            pltpu.SemaphoreType.DMA,
            pltpu.SemaphoreType.DMA,
        ],
    )
    def seg(hs_hbm, src_hbm, dst_hbm, z_hbm, out_hbm,
            sidx, didx, rows0, rows1, acc, gsem0, gsem1, isem):
        c = lax.axis_index("c")
        s = lax.axis_index("s")
        r0 = s * _RPT
        pltpu.sync_copy(z_hbm.at[pl.ds(r0, _RPT)], acc.at[pl.ds(r0, _RPT)])

        @pl.when(s == _NS - 1)
        def _():
            pltpu.sync_copy(z_hbm.at[pl.ds(_NS * _RPT, _N - _NS * _RPT)],
                            acc.at[pl.ds(_NS * _RPT, _N - _NS * _RPT)])

        sbase = c * cpc + s * cpt
        dbase = (c * cpc + s * cpt) if edge_split else (s * cpt)
        plsc.subcore_barrier()

        rows = (rows0, rows1)
        sems = (gsem0, gsem1)

        @pl.loop(0, cpt // ob)
        def _(t):
            ip = pltpu.async_copy(src_hbm.at[pl.ds(sbase + t * ob, ob)],
                                  sidx, isem)
            pltpu.sync_copy(dst_hbm.at[pl.ds(dbase + t * ob, ob)], didx)
            ip.wait()
            pend = [pltpu.async_copy(hs_hbm.at[sidx.at[0]], rows[0], sems[0]),
                    None]
            for j in range(ob):
                if j + 1 < ob:
                    b = (j + 1) % 2
                    pend[b] = pltpu.async_copy(hs_hbm.at[sidx.at[j + 1]],
                                               rows[b], sems[b])
                pend[j % 2].wait()
                pltpu.sync_copy(rows[j % 2], acc.at[didx.at[j]], add=True)

        plsc.subcore_barrier()
        pltpu.sync_copy(acc.at[pl.ds(r0, _RPT)],
                        out_hbm.at[pl.ds(c * _N + r0, _RPT)])

        @pl.when(s == _NS - 1)
        def _():
            pltpu.sync_copy(
                acc.at[pl.ds(_NS * _RPT, _N - _NS * _RPT)],
                out_hbm.at[pl.ds(c * _N + _NS * _RPT, _N - _NS * _RPT)])

    def call(hs, src2, dst2, z):
        assert hs.shape == (hs_rows, 128)
        return seg(hs, src2, dst2, z)

    return call


def _tc_abs(x):
    def body(x_ref, o_ref):
        o_ref[...] = jnp.abs(x_ref[...])

    return pl.pallas_call(
        body,
        grid=(_NBLK,),
        in_specs=[pl.BlockSpec((_BN, 128), lambda i: (i, 0))],
        out_specs=pl.BlockSpec((_BN, 128), lambda i: (i, 0)),
        out_shape=jax.ShapeDtypeStruct((_N, 128), jnp.float32),
    )(x)


def _tc_layer0(aggp, h0, wd, w):
    def body(alo, ahi, h_ref, wd_ref, w_ref, o_ref):
        acc = lax.dot_general(alo[...] + ahi[...], wd_ref[...], _DN, **_DOT_KW)
        acc += lax.dot_general(h_ref[...], w_ref[...], _DN, **_DOT_KW)
        o_ref[...] = jnp.maximum(acc, 0.0)

    return pl.pallas_call(
        body,
        grid=(2, _NBLK),
        in_specs=[
            pl.BlockSpec((_BN, 128), lambda j, i: (i, 0)),
            pl.BlockSpec((_BN, 128), lambda j, i: (_NBLK + i, 0)),
            pl.BlockSpec((_BN, 128), lambda j, i: (i, 0)),
            pl.BlockSpec((128, 128), lambda j, i: (0, j)),
            pl.BlockSpec((128, 128), lambda j, i: (0, j)),
        ],
        out_specs=pl.BlockSpec((_BN, 128), lambda j, i: (j * _NBLK + i, 0)),
        out_shape=jax.ShapeDtypeStruct((2 * _N, 128), jnp.float32),
    )(aggp, aggp, h0, wd, w)


def _tc_layer(aggs, hs, wd, w):
    def body(alo, ahi, hlo, hhi, wd_ref, w_ref, o_ref):
        acc = lax.dot_general(alo[...], wd_ref[pl.ds(0, 128), :],
                              _DN, **_DOT_KW)
        acc += lax.dot_general(ahi[...], wd_ref[pl.ds(128, 128), :],
                               _DN, **_DOT_KW)
        acc += lax.dot_general(hlo[...], w_ref[pl.ds(0, 128), :],
                               _DN, **_DOT_KW)
        acc += lax.dot_general(hhi[...], w_ref[pl.ds(128, 128), :],
                               _DN, **_DOT_KW)
        o_ref[...] = jnp.maximum(acc, 0.0)

    return pl.pallas_call(
        body,
        grid=(2, _NBLK),
        in_specs=[
            pl.BlockSpec((_BN, 128), lambda j, i: (i, 0)),
            pl.BlockSpec((_BN, 128), lambda j, i: (_NBLK + i, 0)),
            pl.BlockSpec((_BN, 128), lambda j, i: (i, 0)),
            pl.BlockSpec((_BN, 128), lambda j, i: (_NBLK + i, 0)),
            pl.BlockSpec((256, 128), lambda j, i: (0, j)),
            pl.BlockSpec((256, 128), lambda j, i: (0, j)),
        ],
        out_specs=pl.BlockSpec((_BN, 128), lambda j, i: (j * _NBLK + i, 0)),
        out_shape=jax.ShapeDtypeStruct((2 * _N, 128), jnp.float32),
    )(aggs, aggs, hs, hs, wd, w)


def _tc_pool(hs3, batch2d):
    def body(h_ref, b_ref, o_ref):
        i = pl.program_id(1)

        @pl.when(i == 0)
        def _():
            o_ref[...] = jnp.zeros_like(o_ref)

        oh = (b_ref[...] == lax.broadcasted_iota(jnp.int32, (_BN, 64), 1))
        o_ref[...] += lax.dot_general(oh.astype(jnp.float32), h_ref[...],
                                      (((0,), (0,)), ((), ())), **_DOT_KW)

    return pl.pallas_call(
        body,
        grid=(2, _NBLK),
        in_specs=[
            pl.BlockSpec((_BN, 128), lambda j, i: (j * _NBLK + i, 0)),
            pl.BlockSpec((_BN, 1), lambda j, i: (i, 0)),
        ],
        out_specs=pl.BlockSpec((64, 128), lambda j, i: (0, j)),
        out_shape=jax.ShapeDtypeStruct((64, 256), jnp.float32),
    )(hs3, batch2d)


def _tc_head(pooled, w1, b1, w2, b2):
    def body(p_ref, w1_ref, b1_ref, w2_ref, b2_ref, o_ref):
        t = lax.dot_general(p_ref[...], w1_ref[...], _DN,
                            **_DOT_KW) + b1_ref[...]
        t = jnp.maximum(t, 0.0)
        o_ref[...] = lax.dot_general(t, w2_ref[...], _DN,
                                     **_DOT_KW) + b2_ref[...]

    return pl.pallas_call(
        body,
        out_shape=jax.ShapeDtypeStruct((64, 10), jnp.float32),
    )(pooled, w1, b1, w2, b2)


def kernel(x, lower_index, batch, W_down_0, W_0, W_down_1, W_1,
           W_down_2, W_2, lin1_w, lin1_b, lin2_w, lin2_b):
    src = lower_index[0]
    dst = lower_index[1]
    src2 = jnp.concatenate([src, src + _N]).reshape(2 * _CHUNKS, _CH)
    dst2 = dst.reshape(_CHUNKS, _CH)
    z128 = jnp.zeros((_N, 128), jnp.float32)
    batch2d = batch.reshape(_N, 1)
    b1 = lin1_b.reshape(1, 256)
    b2 = lin2_b.reshape(1, 10)

    seg_e = _sc_segment_sum(_N, edge_split=True)
    seg_f = _sc_segment_sum(2 * _N, edge_split=False)

    h0 = _tc_abs(x)
    a0 = seg_e(h0, src2, dst2, z128)
    hs1 = _tc_layer0(a0, h0, W_down_0, W_0)
    a1 = seg_f(hs1, src2, dst2, z128)
    hs2 = _tc_layer(a1, hs1, W_down_1, W_1)
    a2 = seg_f(hs2, src2, dst2, z128)
    hs3 = _tc_layer(a2, hs2, W_down_2, W_2)
    pooled = _tc_pool(hs3, batch2d)
    return _tc_head(pooled, lin1_w, b1, lin2_w, b2)


# speedup vs baseline: 9.0751x; 1.0805x over previous
import functools

import jax
import jax.numpy as jnp
from jax import lax
from jax.experimental import pallas as pl
from jax.experimental.pallas import tpu as pltpu
from jax.experimental.pallas import tpu_sc as plsc

_N = 10000
_E = 320000
_NC = 2
_NS = 16
_CH = 125
_OB = 16
_RPT = 624
_CHUNKS = _E // _CH
_BN = 2000
_NBLK = _N // _BN

_DOT_KW = dict(preferred_element_type=jnp.float32,
               precision=lax.Precision.DEFAULT)
_DN = (((1,), (0,)), ((), ()))


def _sc_segment_sum(hs_rows, edge_split):
    cpc = _CHUNKS // 2 if edge_split else _CHUNKS
    cpt = cpc // _NS
    ob = 16 if edge_split else 32
    mesh = plsc.VectorSubcoreMesh(core_axis_name="c", subcore_axis_name="s")

    @functools.partial(
        pl.kernel,
        out_type=jax.ShapeDtypeStruct((2 * _N, 128), jnp.float32),
        mesh=mesh,
        scratch_types=[
            pltpu.VMEM((ob, _CH), jnp.int32),
            pltpu.VMEM((ob, _CH), jnp.int32),
            pltpu.VMEM((_CH, 128), jnp.float32),
            pltpu.VMEM((_CH, 128), jnp.float32),
            pltpu.VMEM_SHARED((_N, 128), jnp.float32),
            pltpu.SemaphoreType.DMA,
            pltpu.SemaphoreType.DMA,
            pltpu.SemaphoreType.DMA,
        ],
    )
    def seg(hs_hbm, src_hbm, dst_hbm, z_hbm, out_hbm,
            sidx, didx, rows0, rows1, acc, gsem0, gsem1, isem):
        c = lax.axis_index("c")
        s = lax.axis_index("s")
        r0 = s * _RPT
        pltpu.sync_copy(z_hbm.at[pl.ds(r0, _RPT)], acc.at[pl.ds(r0, _RPT)])

        @pl.when(s == _NS - 1)
        def _():
            pltpu.sync_copy(z_hbm.at[pl.ds(_NS * _RPT, _N - _NS * _RPT)],
                            acc.at[pl.ds(_NS * _RPT, _N - _NS * _RPT)])

        sbase = c * cpc + s * cpt
        dbase = (c * cpc + s * cpt) if edge_split else (s * cpt)
        plsc.subcore_barrier()

        rows = (rows0, rows1)
        sems = (gsem0, gsem1)

        @pl.loop(0, cpt // ob)
        def _(t):
            ip = pltpu.async_copy(src_hbm.at[pl.ds(sbase + t * ob, ob)],
                                  sidx, isem)
            pltpu.sync_copy(dst_hbm.at[pl.ds(dbase + t * ob, ob)], didx)
            ip.wait()
            pend = [pltpu.async_copy(hs_hbm.at[sidx.at[0]], rows[0], sems[0]),
                    None]
            for j in range(ob):
                if j + 1 < ob:
                    b = (j + 1) % 2
                    pend[b] = pltpu.async_copy(hs_hbm.at[sidx.at[j + 1]],
                                               rows[b], sems[b])
                pend[j % 2].wait()
                pltpu.sync_copy(rows[j % 2], acc.at[didx.at[j]], add=True)

        plsc.subcore_barrier()
        pltpu.sync_copy(acc.at[pl.ds(r0, _RPT)],
                        out_hbm.at[pl.ds(c * _N + r0, _RPT)])

        @pl.when(s == _NS - 1)
        def _():
            pltpu.sync_copy(
                acc.at[pl.ds(_NS * _RPT, _N - _NS * _RPT)],
                out_hbm.at[pl.ds(c * _N + _NS * _RPT, _N - _NS * _RPT)])

    def call(hs, src2, dst2, z):
        assert hs.shape == (hs_rows, 128)
        return seg(hs, src2, dst2, z)

    return call


def _tc_abs(x):
    def body(x_ref, o_ref):
        o_ref[...] = jnp.abs(x_ref[...])

    return pl.pallas_call(
        body,
        grid=(_NBLK,),
        in_specs=[pl.BlockSpec((_BN, 128), lambda i: (i, 0))],
        out_specs=pl.BlockSpec((_BN, 128), lambda i: (i, 0)),
        out_shape=jax.ShapeDtypeStruct((_N, 128), jnp.float32),
    )(x)


def _tc_layer0(aggp, h0, wd, w):
    def body(alo, ahi, h_ref, wd_ref, w_ref, o_ref):
        acc = lax.dot_general(alo[...] + ahi[...], wd_ref[...], _DN, **_DOT_KW)
        acc += lax.dot_general(h_ref[...], w_ref[...], _DN, **_DOT_KW)
        o_ref[...] = jnp.maximum(acc, 0.0)

    return pl.pallas_call(
        body,
        grid=(2, _NBLK),
        in_specs=[
            pl.BlockSpec((_BN, 128), lambda j, i: (i, 0)),
            pl.BlockSpec((_BN, 128), lambda j, i: (_NBLK + i, 0)),
            pl.BlockSpec((_BN, 128), lambda j, i: (i, 0)),
            pl.BlockSpec((128, 128), lambda j, i: (0, j)),
            pl.BlockSpec((128, 128), lambda j, i: (0, j)),
        ],
        out_specs=pl.BlockSpec((_BN, 128), lambda j, i: (j * _NBLK + i, 0)),
        out_shape=jax.ShapeDtypeStruct((2 * _N, 128), jnp.float32),
    )(aggp, aggp, h0, wd, w)


def _tc_layer(aggs, hs, wd, w):
    def body(alo, ahi, hlo, hhi, wd_ref, w_ref, o_ref):
        acc = lax.dot_general(alo[...], wd_ref[pl.ds(0, 128), :],
                              _DN, **_DOT_KW)
        acc += lax.dot_general(ahi[...], wd_ref[pl.ds(128, 128), :],
                               _DN, **_DOT_KW)
        acc += lax.dot_general(hlo[...], w_ref[pl.ds(0, 128), :],
                               _DN, **_DOT_KW)
        acc += lax.dot_general(hhi[...], w_ref[pl.ds(128, 128), :],
                               _DN, **_DOT_KW)
        o_ref[...] = jnp.maximum(acc, 0.0)

    return pl.pallas_call(
        body,
        grid=(2, _NBLK),
        in_specs=[
            pl.BlockSpec((_BN, 128), lambda j, i: (i, 0)),
            pl.BlockSpec((_BN, 128), lambda j, i: (_NBLK + i, 0)),
            pl.BlockSpec((_BN, 128), lambda j, i: (i, 0)),
            pl.BlockSpec((_BN, 128), lambda j, i: (_NBLK + i, 0)),
            pl.BlockSpec((256, 128), lambda j, i: (0, j)),
            pl.BlockSpec((256, 128), lambda j, i: (0, j)),
        ],
        out_specs=pl.BlockSpec((_BN, 128), lambda j, i: (j * _NBLK + i, 0)),
        out_shape=jax.ShapeDtypeStruct((2 * _N, 128), jnp.float32),
    )(aggs, aggs, hs, hs, wd, w)


def _tc_pool_head(hs3, batch2d, w1, b1, w2, b2):
    def body(h_ref, b_ref, w1_ref, b1_ref, w2_ref, b2_ref, o_ref, pool_scr):
        j = pl.program_id(0)
        i = pl.program_id(1)

        @pl.when((j == 0) & (i == 0))
        def _():
            pool_scr[...] = jnp.zeros_like(pool_scr)

        oh = (b_ref[...] == lax.broadcasted_iota(jnp.int32, (_BN, 64), 1))
        pool_scr[j] += lax.dot_general(oh.astype(jnp.float32), h_ref[...],
                                       (((0,), (0,)), ((), ())), **_DOT_KW)

        @pl.when((j == 1) & (i == _NBLK - 1))
        def _():
            p = jnp.concatenate([pool_scr[0], pool_scr[1]], axis=1)
            t = lax.dot_general(p, w1_ref[...], _DN, **_DOT_KW) + b1_ref[...]
            t = jnp.maximum(t, 0.0)
            o_ref[...] = lax.dot_general(t, w2_ref[...], _DN,
                                         **_DOT_KW) + b2_ref[...]

    return pl.pallas_call(
        body,
        grid=(2, _NBLK),
        in_specs=[
            pl.BlockSpec((_BN, 128), lambda j, i: (j * _NBLK + i, 0)),
            pl.BlockSpec((_BN, 1), lambda j, i: (i, 0)),
            pl.BlockSpec((256, 256), lambda j, i: (0, 0)),
            pl.BlockSpec((1, 256), lambda j, i: (0, 0)),
            pl.BlockSpec((256, 10), lambda j, i: (0, 0)),
            pl.BlockSpec((1, 10), lambda j, i: (0, 0)),
        ],
        out_specs=pl.BlockSpec((64, 10), lambda j, i: (0, 0)),
        out_shape=jax.ShapeDtypeStruct((64, 10), jnp.float32),
        scratch_shapes=[pltpu.VMEM((2, 64, 128), jnp.float32)],
    )(hs3, batch2d, w1, b1, w2, b2)


def kernel(x, lower_index, batch, W_down_0, W_0, W_down_1, W_1,
           W_down_2, W_2, lin1_w, lin1_b, lin2_w, lin2_b):
    src = lower_index[0]
    dst = lower_index[1]
    src2 = jnp.concatenate([src, src + _N]).reshape(2 * _CHUNKS, _CH)
    dst2 = dst.reshape(_CHUNKS, _CH)
    z128 = jnp.zeros((_N, 128), jnp.float32)
    batch2d = batch.reshape(_N, 1)
    b1 = lin1_b.reshape(1, 256)
    b2 = lin2_b.reshape(1, 10)

    seg_e = _sc_segment_sum(_N, edge_split=True)
    seg_f = _sc_segment_sum(2 * _N, edge_split=False)

    h0 = _tc_abs(x)
    a0 = seg_e(h0, src2, dst2, z128)
    hs1 = _tc_layer0(a0, h0, W_down_0, W_0)
    a1 = seg_f(hs1, src2, dst2, z128)
    hs2 = _tc_layer(a1, hs1, W_down_1, W_1)
    a2 = seg_f(hs2, src2, dst2, z128)
    hs3 = _tc_layer(a2, hs2, W_down_2, W_2)
    return _tc_pool_head(hs3, batch2d, lin1_w, b1, lin2_w, b2)
